# baseline (speedup 1.0000x reference)
import functools

import jax
import jax.numpy as jnp
from jax import lax
from jax.experimental import pallas as pl
from jax.experimental.pallas import tpu as pltpu
from jax.experimental.pallas import tpu_sc as plsc

VOCAB = 100000
HIDDEN = 768
EPS = 1e-5
LANES = 16
NJ = HIDDEN // LANES

CHUNK = 32


def _row_body(buf, gam_v, bet_v, r):
    s = jnp.zeros((LANES,), jnp.float32)
    s2 = jnp.zeros((LANES,), jnp.float32)
    for j in range(NJ):
        v = buf[r, pl.ds(j * LANES, LANES)]
        s = s + v
        s2 = s2 + v * v
    tot = jnp.sum(s)
    tot2 = jnp.sum(s2)
    mean = tot * (1.0 / HIDDEN)
    var = tot2 * (1.0 / HIDDEN) - mean * mean
    x = var + EPS
    i = lax.bitcast_convert_type(x, jnp.int32)
    i = jnp.int32(0x5F3759DF) - lax.shift_right_logical(i, 1)
    y = lax.bitcast_convert_type(i, jnp.float32)
    for _ in range(3):
        y = y * (1.5 - 0.5 * x * y * y)
    rstd = y
    for j in range(NJ):
        sl = pl.ds(j * LANES, LANES)
        v = buf[r, sl]
        g = gam_v[sl]
        b = bet_v[sl]
        buf[r, sl] = (v - mean) * (rstd * g) + b
    return r


def _make_kernel(n_rows):
    info = plsc.get_sparse_core_info()
    nw = info.num_cores * info.num_subcores
    bpw = n_rows // nw
    nchunk = bpw // CHUNK
    mesh = plsc.VectorSubcoreMesh(core_axis_name="c", subcore_axis_name="s")

    @functools.partial(
        pl.kernel,
        out_type=jax.ShapeDtypeStruct((n_rows, HIDDEN), jnp.float32),
        mesh=mesh,
        compiler_params=pltpu.CompilerParams(needs_layout_passes=False),
        scratch_types=[
            pltpu.VMEM((bpw,), jnp.int32),
            pltpu.VMEM((CHUNK, HIDDEN), jnp.float32),
            pltpu.VMEM((CHUNK, HIDDEN), jnp.float32),
            pltpu.VMEM((HIDDEN,), jnp.float32),
            pltpu.VMEM((HIDDEN,), jnp.float32),
            pltpu.SemaphoreType.DMA,
            pltpu.SemaphoreType.DMA,
            pltpu.SemaphoreType.DMA,
            pltpu.SemaphoreType.DMA,
        ],
    )
    def kern(ids_hbm, table_hbm, gamma_hbm, beta_hbm, out_hbm,
             idx_v, buf0, buf1, gam_v, bet_v, g0, g1, o0, o1):
        wid = lax.axis_index("s") * info.num_cores + lax.axis_index("c")
        base = wid * bpw
        bufs = (buf0, buf1)
        gsems = (g0, g1)
        osems = (o0, o1)

        pltpu.sync_copy(ids_hbm.at[pl.ds(base, bpw)], idx_v)
        pltpu.sync_copy(gamma_hbm, gam_v)
        pltpu.sync_copy(beta_hbm, bet_v)
        pltpu.async_copy(table_hbm.at[idx_v.at[pl.ds(0, CHUNK)]], buf0, g0)

        def outer(g, carry):
            for b in (0, 1):
                c = 2 * g + b
                nb = 1 - b
                cond_wait = (g >= 1) if b == 0 else None
                cond_pref = None if b == 0 else (g < nchunk // 2 - 1)

                def wait_prev_store():
                    pltpu.make_async_copy(
                        bufs[nb],
                        out_hbm.at[pl.ds(base + (c - 1) * CHUNK, CHUNK)],
                        osems[nb]).wait()

                def prefetch_next():
                    pltpu.async_copy(
                        table_hbm.at[idx_v.at[pl.ds((c + 1) * CHUNK, CHUNK)]],
                        bufs[nb], gsems[nb])

                if cond_wait is None:
                    wait_prev_store()
                else:
                    pl.when(cond_wait)(wait_prev_store)
                if cond_pref is None:
                    prefetch_next()
                else:
                    pl.when(cond_pref)(prefetch_next)

                pltpu.make_async_copy(
                    table_hbm.at[idx_v.at[pl.ds(c * CHUNK, CHUNK)]],
                    bufs[b], gsems[b]).wait()
                lax.fori_loop(
                    0, CHUNK,
                    lambda r, _, _b=b: _row_body(bufs[_b], gam_v, bet_v, r),
                    0)
                pltpu.async_copy(
                    bufs[b], out_hbm.at[pl.ds(base + c * CHUNK, CHUNK)],
                    osems[b])
            return carry

        lax.fori_loop(0, nchunk // 2, outer, 0)
        pltpu.make_async_copy(
            buf1, out_hbm.at[pl.ds(base + (nchunk - 1) * CHUNK, CHUNK)],
            o1).wait()

    return kern


def kernel(input_ids, table, gamma, beta):
    bsz, seq = input_ids.shape
    n_rows = bsz * seq
    ids_flat = input_ids.reshape(n_rows)
    out = _make_kernel(n_rows)(ids_flat, table, gamma, beta)
    return out.reshape(bsz, seq, HIDDEN)
